# baseline (speedup 1.0000x reference)
import jax
import jax.numpy as jnp
from jax import lax
from jax.experimental import pallas as pl
from jax.experimental.pallas import tpu as pltpu

F = 16
H = 768
P = 196
NV = 16
NQ = NV * P


def _body(in_hbm, emb_ref, out_hbm, in_st, vbuf, gsems, wsems):
    b = pl.program_id(0)
    g = b % 2

    def start_in(video, grp):
        for f in range(F):
            pltpu.make_async_copy(
                in_hbm.at[video * F + f], in_st.at[grp, f], gsems.at[grp, f]
            ).start()

    @pl.when(b == 0)
    def _():
        start_in(0, 0)
        start_in(1, 1)

    @pl.when(b >= 2)
    def _():
        pltpu.make_async_copy(vbuf.at[g], out_hbm.at[b - 2], wsems.at[g]).wait()

    for f in range(F):
        pltpu.make_async_copy(
            in_hbm.at[b * F + f], in_st.at[g, f], gsems.at[g, f]
        ).wait()
        vbuf[g, :, f * H:(f + 1) * H] = in_st[g, f] + emb_ref[pl.ds(f, 1)]

    pltpu.make_async_copy(vbuf.at[g], out_hbm.at[b], wsems.at[g]).start()

    @pl.when(b + 2 < NV)
    def _():
        start_in(b + 2, g)

    @pl.when(b == NV - 1)
    def _():
        for d in range(2):
            pltpu.make_async_copy(
                vbuf.at[d], out_hbm.at[NV - 2 + d], wsems.at[d]
            ).wait()


@jax.jit
def _tc_call(in3, emb_table):
    return pl.pallas_call(
        _body,
        grid=(NV,),
        in_specs=[
            pl.BlockSpec(memory_space=pltpu.HBM),
            pl.BlockSpec((F, H), lambda i: (0, 0)),
        ],
        out_specs=pl.BlockSpec(memory_space=pltpu.HBM),
        out_shape=jax.ShapeDtypeStruct((NV, P, F * H), jnp.float32),
        scratch_shapes=[
            pltpu.VMEM((2, F, P, H), jnp.float32),
            pltpu.VMEM((2, P, F * H), jnp.float32),
            pltpu.SemaphoreType.DMA((2, F)),
            pltpu.SemaphoreType.DMA((2,)),
        ],
    )(in3, emb_table)


def kernel(inputs, emb_table):
    out = _tc_call(inputs, emb_table)
    return out.reshape(NQ, F, H)


# speedup vs baseline: 1.8813x; 1.8813x over previous
import jax
import jax.numpy as jnp
from jax import lax
from jax.experimental import pallas as pl
from jax.experimental.pallas import tpu as pltpu

F = 16
H = 768
P = 196
NV = 16
NQ = NV * P
NT = NV * F

K = 24
LA = 20


def _body(in_hbm, emb_ref, out_hbm, in_st, out_st, gsems, wsems):
    i = pl.program_id(0)
    b = i // F
    f = i % F

    def start_in(t):
        pltpu.make_async_copy(in_hbm.at[t], in_st.at[t % K], gsems.at[t % K]).start()

    @pl.when(i == 0)
    def _():
        for t in range(LA + 1):
            start_in(t)

    @pl.when((i + LA < NT) & (i > 0))
    def _():
        start_in(i + LA)

    pltpu.make_async_copy(in_hbm.at[i], in_st.at[i % K], gsems.at[i % K]).wait()

    @pl.when(i >= K)
    def _():
        pltpu.make_async_copy(
            out_st.at[i % K], out_hbm.at[pl.ds(0, P), 0], wsems.at[i % K]
        ).wait()

    out_st[i % K] = in_st[i % K] + emb_ref[pl.ds(f, 1)]

    pltpu.make_async_copy(
        out_st.at[i % K], out_hbm.at[pl.ds(b * P, P), f], wsems.at[i % K]
    ).start()

    @pl.when(i == NT - 1)
    def _():
        for d in range(K):
            pltpu.make_async_copy(
                out_st.at[d], out_hbm.at[pl.ds(0, P), 0], wsems.at[d]
            ).wait()


@jax.jit
def _tc_call(in3, emb_table):
    return pl.pallas_call(
        _body,
        grid=(NT,),
        in_specs=[
            pl.BlockSpec(memory_space=pltpu.HBM),
            pl.BlockSpec((F, H), lambda i: (0, 0)),
        ],
        out_specs=pl.BlockSpec(memory_space=pltpu.HBM),
        out_shape=jax.ShapeDtypeStruct((NQ, F, H), jnp.float32),
        scratch_shapes=[
            pltpu.VMEM((K, P, H), jnp.float32),
            pltpu.VMEM((K, P, H), jnp.float32),
            pltpu.SemaphoreType.DMA((K,)),
            pltpu.SemaphoreType.DMA((K,)),
        ],
    )(in3, emb_table)


def kernel(inputs, emb_table):
    return _tc_call(inputs, emb_table)


# speedup vs baseline: 1.8841x; 1.0015x over previous
import jax
import jax.numpy as jnp
from jax.experimental import pallas as pl
from jax.experimental.pallas import tpu as pltpu

F = 16
H = 768
P = 196
NV = 16
NQ = NV * P
NT = NV * F

K = 16
LA = 12


def _body(in_hbm, emb_ref, out_hbm, in_st, out_st, gsems, wsems):
    i = pl.program_id(0)
    b = i // F
    f = i % F

    def start_in(t):
        pltpu.make_async_copy(in_hbm.at[t], in_st.at[t % K], gsems.at[t % K]).start()

    @pl.when(i == 0)
    def _():
        for t in range(LA + 1):
            start_in(t)

    @pl.when((i + LA < NT) & (i > 0))
    def _():
        start_in(i + LA)

    pltpu.make_async_copy(in_hbm.at[i], in_st.at[i % K], gsems.at[i % K]).wait()

    @pl.when(i >= K)
    def _():
        pltpu.make_async_copy(
            out_st.at[i % K], out_hbm.at[pl.ds(0, P), 0], wsems.at[i % K]
        ).wait()

    out_st[i % K] = in_st[i % K] + emb_ref[pl.ds(f, 1)]

    pltpu.make_async_copy(
        out_st.at[i % K], out_hbm.at[pl.ds(b * P, P), f], wsems.at[i % K]
    ).start()

    @pl.when(i == NT - 1)
    def _():
        for d in range(K):
            pltpu.make_async_copy(
                out_st.at[d], out_hbm.at[pl.ds(0, P), 0], wsems.at[d]
            ).wait()


@jax.jit
def _tc_call(in3, emb_table):
    return pl.pallas_call(
        _body,
        grid=(NT,),
        in_specs=[
            pl.BlockSpec(memory_space=pltpu.HBM),
            pl.BlockSpec((F, H), lambda i: (0, 0)),
        ],
        out_specs=pl.BlockSpec(memory_space=pltpu.HBM),
        out_shape=jax.ShapeDtypeStruct((NQ, F, H), jnp.float32),
        scratch_shapes=[
            pltpu.VMEM((K, P, H), jnp.float32),
            pltpu.VMEM((K, P, H), jnp.float32),
            pltpu.SemaphoreType.DMA((K,)),
            pltpu.SemaphoreType.DMA((K,)),
        ],
    )(in3, emb_table)


def kernel(inputs, emb_table):
    return _tc_call(inputs, emb_table)
